# speedup vs baseline: 5.8247x; 5.8247x over previous
import functools

import jax
import jax.numpy as jnp
from jax import lax
from jax.experimental import pallas as pl
from jax.experimental.pallas import tpu as pltpu
from jax.experimental.pallas import tpu_sc as plsc

NUM_WORKERS = 32
CHUNK = 128


@functools.cache
def _build(n_total, n_chunks, embed_dim):
    mesh = plsc.VectorSubcoreMesh(core_axis_name="c", subcore_axis_name="s")

    @functools.partial(
        pl.kernel,
        mesh=mesh,
        out_type=jax.ShapeDtypeStruct((n_total, embed_dim), jnp.float32),
        scratch_types=[
            pltpu.VMEM((n_chunks, CHUNK), jnp.int32),
            pltpu.VMEM((CHUNK, embed_dim), jnp.float32),
            pltpu.SemaphoreType.DMA,
        ],
    )
    def gather_kernel(idx_hbm, table_hbm, out_hbm, idx_v, rows_v, sem):
        wid = lax.axis_index("s") * 2 + lax.axis_index("c")
        base = wid * (n_chunks * CHUNK)
        pltpu.sync_copy(idx_hbm.at[wid], idx_v)

        def body(i, carry):
            pltpu.async_copy(table_hbm.at[idx_v.at[i]], rows_v, sem).wait()
            pltpu.sync_copy(rows_v, out_hbm.at[pl.ds(base + i * CHUNK, CHUNK)])
            return carry

        lax.fori_loop(0, n_chunks, body, 0)

    return gather_kernel


def kernel(position_ids, table):
    n_total = position_ids.size
    embed_dim = table.shape[1]
    n_chunks = n_total // (NUM_WORKERS * CHUNK)
    idx3 = position_ids.reshape(NUM_WORKERS, n_chunks, CHUNK)
    out = _build(n_total, n_chunks, embed_dim)(idx3, table)
    return out.reshape(position_ids.shape + (embed_dim,))


# speedup vs baseline: 7.8983x; 1.3560x over previous
import functools

import jax
import jax.numpy as jnp
from jax import lax
from jax.experimental import pallas as pl
from jax.experimental.pallas import tpu as pltpu
from jax.experimental.pallas import tpu_sc as plsc

NUM_WORKERS = 32
CHUNK = 128
NBUF = 4


@functools.cache
def _build(n_total, n_chunks, embed_dim):
    mesh = plsc.VectorSubcoreMesh(core_axis_name="c", subcore_axis_name="s")

    @functools.partial(
        pl.kernel,
        mesh=mesh,
        out_type=jax.ShapeDtypeStruct((n_total, embed_dim), jnp.float32),
        scratch_types=[
            pltpu.VMEM((n_chunks, CHUNK), jnp.int32),
            pltpu.VMEM((NBUF, CHUNK, embed_dim), jnp.float32),
            pltpu.SemaphoreType.DMA,
            pltpu.SemaphoreType.DMA,
        ],
    )
    def gather_kernel(idx_hbm, table_hbm, out_hbm, idx_v, rows_v, gsem, osem):
        wid = lax.axis_index("s") * 2 + lax.axis_index("c")
        base = wid * (n_chunks * CHUNK)
        pltpu.sync_copy(idx_hbm.at[wid], idx_v)

        def start_gather(buf, chunk):
            pltpu.async_copy(table_hbm.at[idx_v.at[chunk]], rows_v.at[buf], gsem)

        for b in range(NBUF):
            start_gather(b, b)

        def body(j, carry):
            first = j * NBUF
            for b in range(NBUF):
                pltpu.make_async_copy(
                    table_hbm.at[idx_v.at[first + b]], rows_v.at[b], gsem
                ).wait()
                pltpu.async_copy(
                    rows_v.at[b],
                    out_hbm.at[pl.ds(base + (first + b) * CHUNK, CHUNK)],
                    osem,
                )
            for b in range(NBUF):
                nxt = first + NBUF + b
                pltpu.make_async_copy(
                    rows_v.at[b],
                    out_hbm.at[pl.ds(base, CHUNK)],
                    osem,
                ).wait()

                @pl.when(nxt < n_chunks)
                def _():
                    start_gather(b, nxt)

            return carry

        lax.fori_loop(0, n_chunks // NBUF, body, 0)

    return gather_kernel


def kernel(position_ids, table):
    n_total = position_ids.size
    embed_dim = table.shape[1]
    n_chunks = n_total // (NUM_WORKERS * CHUNK)
    idx3 = position_ids.reshape(NUM_WORKERS, n_chunks, CHUNK)
    out = _build(n_total, n_chunks, embed_dim)(idx3, table)
    return out.reshape(position_ids.shape + (embed_dim,))


# speedup vs baseline: 7.9163x; 1.0023x over previous
import functools

import jax
import jax.numpy as jnp
from jax import lax
from jax.experimental import pallas as pl
from jax.experimental.pallas import tpu as pltpu
from jax.experimental.pallas import tpu_sc as plsc

NUM_WORKERS = 32
CHUNK = 128
NBUF = 4


@functools.cache
def _build(n_total, n_chunks, embed_dim, n_rows):
    mesh = plsc.VectorSubcoreMesh(core_axis_name="c", subcore_axis_name="s")
    n_sub = 16
    rows_per_sub = (n_rows - 1) // n_sub
    rem_base = rows_per_sub * n_sub
    rem = n_rows - rem_base

    @functools.partial(
        pl.kernel,
        mesh=mesh,
        out_type=jax.ShapeDtypeStruct((n_total, embed_dim), jnp.float32),
        scratch_types=[
            pltpu.VMEM((n_chunks, CHUNK), jnp.int32),
            pltpu.VMEM((NBUF, CHUNK, embed_dim), jnp.float32),
            pltpu.SemaphoreType.DMA,
            pltpu.SemaphoreType.DMA,
        ],
    )
    def gather_kernel(idx_hbm, table_hbm, out_hbm, idx_v, rows_v,
                      gsem, osem):
        sid = lax.axis_index("s")
        wid = sid * 2 + lax.axis_index("c")
        base = wid * (n_chunks * CHUNK)
        pltpu.sync_copy(idx_hbm.at[wid], idx_v)

        def start_gather(buf, chunk):
            pltpu.async_copy(table_hbm.at[idx_v.at[chunk]], rows_v.at[buf], gsem)

        for b in range(NBUF):
            start_gather(b, b)

        def body(j, carry):
            first = j * NBUF
            for b in range(NBUF):
                pltpu.make_async_copy(
                    table_hbm.at[idx_v.at[first + b]], rows_v.at[b], gsem
                ).wait()
                pltpu.async_copy(
                    rows_v.at[b],
                    out_hbm.at[pl.ds(base + (first + b) * CHUNK, CHUNK)],
                    osem,
                )
            for b in range(NBUF):
                nxt = first + NBUF + b
                pltpu.make_async_copy(
                    rows_v.at[b],
                    out_hbm.at[pl.ds(base, CHUNK)],
                    osem,
                ).wait()

                @pl.when(nxt < n_chunks)
                def _():
                    start_gather(b, nxt)

            return carry

        lax.fori_loop(0, n_chunks // NBUF, body, 0)

    return gather_kernel


def kernel(position_ids, table):
    n_total = position_ids.size
    embed_dim = table.shape[1]
    n_chunks = n_total // (NUM_WORKERS * CHUNK)
    idx3 = position_ids.reshape(NUM_WORKERS, n_chunks, CHUNK)
    out = _build(n_total, n_chunks, embed_dim, table.shape[0])(idx3, table)
    return out.reshape(position_ids.shape + (embed_dim,))


# speedup vs baseline: 12.0970x; 1.5281x over previous
import functools

import jax
import jax.numpy as jnp
from jax import lax
from jax.experimental import pallas as pl
from jax.experimental.pallas import tpu as pltpu
from jax.experimental.pallas import tpu_sc as plsc

NUM_WORKERS = 32
CHUNK = 128
NBUF = 2


@functools.cache
def _build(n_total, n_chunks, embed_dim, n_rows):
    mesh = plsc.VectorSubcoreMesh(core_axis_name="c", subcore_axis_name="s")
    n_sub = 16
    rows_per_sub = (n_rows - 1) // n_sub
    rem_base = rows_per_sub * n_sub
    rem = n_rows - rem_base

    @functools.partial(
        pl.kernel,
        mesh=mesh,
        out_type=jax.ShapeDtypeStruct((n_total, embed_dim), jnp.float32),
        scratch_types=[
            pltpu.VMEM((n_chunks, CHUNK), jnp.int32),
            pltpu.VMEM((NBUF, CHUNK, embed_dim), jnp.float32),
            pltpu.VMEM_SHARED((n_rows, embed_dim), jnp.float32),
            pltpu.SemaphoreType.DMA,
            pltpu.SemaphoreType.DMA,
        ],
    )
    def gather_kernel(idx_hbm, table_hbm, out_hbm, idx_v, rows_v, table_sp,
                      gsem, osem):
        sid = lax.axis_index("s")
        wid = sid * 2 + lax.axis_index("c")
        base = wid * (n_chunks * CHUNK)
        pltpu.sync_copy(idx_hbm.at[wid], idx_v)

        pltpu.sync_copy(
            table_hbm.at[pl.ds(sid * rows_per_sub, rows_per_sub)],
            table_sp.at[pl.ds(sid * rows_per_sub, rows_per_sub)],
        )

        @pl.when(sid == 0)
        def _():
            pltpu.sync_copy(
                table_hbm.at[pl.ds(rem_base, rem)],
                table_sp.at[pl.ds(rem_base, rem)],
            )

        plsc.subcore_barrier()

        def start_gather(buf, chunk):
            pltpu.async_copy(table_sp.at[idx_v.at[chunk]], rows_v.at[buf], gsem)

        for b in range(NBUF):
            start_gather(b, b)

        def body(j, carry):
            first = j * NBUF
            for b in range(NBUF):
                pltpu.make_async_copy(
                    table_sp.at[idx_v.at[first + b]], rows_v.at[b], gsem
                ).wait()
                pltpu.async_copy(
                    rows_v.at[b],
                    out_hbm.at[pl.ds(base + (first + b) * CHUNK, CHUNK)],
                    osem,
                )
            for b in range(NBUF):
                nxt = first + NBUF + b
                pltpu.make_async_copy(
                    rows_v.at[b],
                    out_hbm.at[pl.ds(base, CHUNK)],
                    osem,
                ).wait()

                @pl.when(nxt < n_chunks)
                def _():
                    start_gather(b, nxt)

            return carry

        lax.fori_loop(0, n_chunks // NBUF, body, 0)

    return gather_kernel


def kernel(position_ids, table):
    n_total = position_ids.size
    embed_dim = table.shape[1]
    n_chunks = n_total // (NUM_WORKERS * CHUNK)
    idx3 = position_ids.reshape(NUM_WORKERS, n_chunks, CHUNK)
    out = _build(n_total, n_chunks, embed_dim, table.shape[0])(idx3, table)
    return out.reshape(position_ids.shape + (embed_dim,))


# speedup vs baseline: 12.1356x; 1.0032x over previous
import functools

import jax
import jax.numpy as jnp
from jax import lax
from jax.experimental import pallas as pl
from jax.experimental.pallas import tpu as pltpu
from jax.experimental.pallas import tpu_sc as plsc

NUM_WORKERS = 32
CHUNK = 64
NBUF = 4


@functools.cache
def _build(n_total, n_chunks, embed_dim, n_rows):
    mesh = plsc.VectorSubcoreMesh(core_axis_name="c", subcore_axis_name="s")
    n_sub = 16
    rows_per_sub = (n_rows - 1) // n_sub
    rem_base = rows_per_sub * n_sub
    rem = n_rows - rem_base

    @functools.partial(
        pl.kernel,
        mesh=mesh,
        out_type=jax.ShapeDtypeStruct((n_total, embed_dim), jnp.float32),
        scratch_types=[
            pltpu.VMEM((n_chunks, CHUNK), jnp.int32),
            pltpu.VMEM((NBUF, CHUNK, embed_dim), jnp.float32),
            pltpu.VMEM_SHARED((n_rows, embed_dim), jnp.float32),
            pltpu.SemaphoreType.DMA,
            pltpu.SemaphoreType.DMA,
        ],
    )
    def gather_kernel(idx_hbm, table_hbm, out_hbm, idx_v, rows_v, table_sp,
                      gsem, osem):
        sid = lax.axis_index("s")
        wid = sid * 2 + lax.axis_index("c")
        base = wid * (n_chunks * CHUNK)
        pltpu.sync_copy(idx_hbm.at[wid], idx_v)

        pltpu.sync_copy(
            table_hbm.at[pl.ds(sid * rows_per_sub, rows_per_sub)],
            table_sp.at[pl.ds(sid * rows_per_sub, rows_per_sub)],
        )

        @pl.when(sid == 0)
        def _():
            pltpu.sync_copy(
                table_hbm.at[pl.ds(rem_base, rem)],
                table_sp.at[pl.ds(rem_base, rem)],
            )

        plsc.subcore_barrier()

        def start_gather(buf, chunk):
            pltpu.async_copy(table_sp.at[idx_v.at[chunk]], rows_v.at[buf], gsem)

        for b in range(NBUF):
            start_gather(b, b)

        def body(j, carry):
            first = j * NBUF
            for b in range(NBUF):
                pltpu.make_async_copy(
                    table_sp.at[idx_v.at[first + b]], rows_v.at[b], gsem
                ).wait()
                pltpu.async_copy(
                    rows_v.at[b],
                    out_hbm.at[pl.ds(base + (first + b) * CHUNK, CHUNK)],
                    osem,
                )
            for b in range(NBUF):
                nxt = first + NBUF + b
                pltpu.make_async_copy(
                    rows_v.at[b],
                    out_hbm.at[pl.ds(base, CHUNK)],
                    osem,
                ).wait()

                @pl.when(nxt < n_chunks)
                def _():
                    start_gather(b, nxt)

            return carry

        lax.fori_loop(0, n_chunks // NBUF, body, 0)

    return gather_kernel


def kernel(position_ids, table):
    n_total = position_ids.size
    embed_dim = table.shape[1]
    n_chunks = n_total // (NUM_WORKERS * CHUNK)
    idx3 = position_ids.reshape(NUM_WORKERS, n_chunks, CHUNK)
    out = _build(n_total, n_chunks, embed_dim, table.shape[0])(idx3, table)
    return out.reshape(position_ids.shape + (embed_dim,))
